# baseline (speedup 1.0000x reference)
import functools
import jax
import jax.numpy as jnp
from jax import lax
from jax.experimental import pallas as pl
from jax.experimental.pallas import tpu as pltpu
from jax.experimental.pallas import tpu_sc as plsc

N_NODES = 10000
N_EDGES = 160000
R_PAD = 10240
E_PAD = 163840
K = 128
NCH_AGG = 80
NCH_CNT = 40
ROWS_PER_TILE = R_PAD // 16

_mesh = plsc.VectorSubcoreMesh(core_axis_name="c", subcore_axis_name="s")


@functools.partial(
    pl.kernel,
    out_type=jax.ShapeDtypeStruct((2, R_PAD, 128), jnp.float32),
    mesh=_mesh,
    scratch_types=[
        pltpu.VMEM((NCH_CNT, K), jnp.int32),
        pltpu.VMEM((K, 128), jnp.float32),
        pltpu.VMEM_SHARED((R_PAD, 128), jnp.float32),
    ],
)
def _sc_count(dst_hbm, ones_hbm, zeros_hbm, out_hbm, dst_v, ones_v, acc):
    cid = lax.axis_index("c")
    sid = lax.axis_index("s")
    wid = sid * 2 + cid
    rows = pl.ds(sid * ROWS_PER_TILE, ROWS_PER_TILE)

    pltpu.sync_copy(zeros_hbm.at[rows], acc.at[rows])
    pltpu.sync_copy(ones_hbm, ones_v)
    pltpu.sync_copy(dst_hbm.at[wid], dst_v)
    plsc.subcore_barrier()

    def body(c, carry):
        pltpu.sync_copy(ones_v, acc.at[dst_v.at[c]], add=True)
        return carry

    lax.fori_loop(0, NCH_CNT, body, 0)
    plsc.subcore_barrier()
    pltpu.sync_copy(acc.at[rows], out_hbm.at[cid, rows])


_NB = 2
_PADC = 8


def _agg_loop(nch, table_hbm, zeros_hbm, src_v, dst_v, bufs, sems, acc):
    for b in range(_NB):
        pltpu.async_copy(table_hbm.at[src_v.at[b]], bufs[b], sems[b])

    def body(i, carry):
        c0 = i * _NB
        for b in range(_NB):
            pltpu.make_async_copy(
                zeros_hbm.at[pl.ds(0, K)], bufs[b], sems[b]).wait()
            pltpu.sync_copy(bufs[b], acc.at[dst_v.at[c0 + b]], add=True)
            pltpu.async_copy(
                table_hbm.at[src_v.at[c0 + b + _NB]], bufs[b], sems[b])
        return carry

    lax.fori_loop(0, nch // _NB, body, 0)
    for b in range(_NB):
        pltpu.make_async_copy(
            zeros_hbm.at[pl.ds(0, K)], bufs[b], sems[b]).wait()


_PH = 2
_CH = NCH_AGG // _PH


@functools.partial(
    pl.kernel,
    out_type=jax.ShapeDtypeStruct((2, R_PAD, 128), jnp.float32),
    mesh=_mesh,
    scratch_types=[
        pltpu.VMEM((_CH + _PADC, K), jnp.int32),
        pltpu.VMEM((_CH, K), jnp.int32),
        pltpu.VMEM((K, 128), jnp.float32),
        pltpu.VMEM((K, 128), jnp.float32),
        pltpu.VMEM_SHARED((R_PAD, 128), jnp.float32),
        pltpu.SemaphoreType.DMA,
        pltpu.SemaphoreType.DMA,
    ],
)
def _sc_agg128(src2_hbm, dst_hbm, table_hbm, zeros_hbm, out_hbm,
               src_v, dst_v, buf0, buf1, acc, sem0, sem1):
    cid = lax.axis_index("c")
    sid = lax.axis_index("s")
    rows = pl.ds(sid * ROWS_PER_TILE, ROWS_PER_TILE)

    pltpu.sync_copy(zeros_hbm.at[rows], acc.at[rows])
    plsc.subcore_barrier()

    def phase(p, carry):
        pltpu.sync_copy(src2_hbm.at[cid, sid, pl.ds(p * _CH, _CH + _PADC)],
                        src_v)
        pltpu.sync_copy(dst_hbm.at[sid, pl.ds(p * _CH, _CH)], dst_v)
        _agg_loop(_CH, table_hbm, zeros_hbm, src_v, dst_v,
                  (buf0, buf1), (sem0, sem1), acc)
        return carry

    lax.fori_loop(0, _PH, phase, 0)
    plsc.subcore_barrier()
    pltpu.sync_copy(acc.at[rows], out_hbm.at[cid, rows])


@functools.partial(
    pl.kernel,
    out_type=jax.ShapeDtypeStruct((2, R_PAD, 128), jnp.float32),
    mesh=_mesh,
    scratch_types=[
        pltpu.VMEM((NCH_CNT + _PADC, K), jnp.int32),
        pltpu.VMEM((NCH_CNT, K), jnp.int32),
        pltpu.VMEM((K, 128), jnp.float32),
        pltpu.VMEM((K, 128), jnp.float32),
        pltpu.VMEM_SHARED((R_PAD, 128), jnp.float32),
        pltpu.SemaphoreType.DMA,
        pltpu.SemaphoreType.DMA,
    ],
)
def _sc_agg_esplit(src_hbm, dst_hbm, table_hbm, zeros_hbm, out_hbm,
                   src_v, dst_v, buf0, buf1, acc, sem0, sem1):
    cid = lax.axis_index("c")
    sid = lax.axis_index("s")
    wid = sid * 2 + cid
    rows = pl.ds(sid * ROWS_PER_TILE, ROWS_PER_TILE)

    pltpu.sync_copy(zeros_hbm.at[rows], acc.at[rows])
    pltpu.sync_copy(src_hbm.at[wid], src_v)
    pltpu.sync_copy(dst_hbm.at[wid], dst_v)
    plsc.subcore_barrier()
    _agg_loop(NCH_CNT, table_hbm, zeros_hbm, src_v, dst_v,
              (buf0, buf1), (sem0, sem1), acc)
    plsc.subcore_barrier()
    pltpu.sync_copy(acc.at[rows], out_hbm.at[cid, rows])


_R = 2000
_NR = N_NODES // _R


def _dinv_block(cnt_blk):
    deg = cnt_blk[0] + cnt_blk[1] + 1.0
    return lax.rsqrt(deg)[:, 0:1]


def _tc_pre_body(x_ref, w_ref, cnt_ref, o_ref):
    dinv = _dinv_block(cnt_ref[...])
    h = jnp.dot(x_ref[...], w_ref[...], preferred_element_type=jnp.float32)
    g = h * dinv
    o_ref[0] = g[:, :128]
    o_ref[1] = g[:, 128:]


def _tc_pre(x, W1, cnt):
    return pl.pallas_call(
        _tc_pre_body,
        grid=(_NR,),
        in_specs=[
            pl.BlockSpec((_R, 256), lambda i: (i, 0)),
            pl.BlockSpec((256, 256), lambda i: (0, 0)),
            pl.BlockSpec((2, _R, 128), lambda i: (0, i, 0)),
        ],
        out_specs=pl.BlockSpec((2, _R, 128), lambda i: (0, i, 0)),
        out_shape=jax.ShapeDtypeStruct((2, N_NODES, 128), jnp.float32),
    )(x, W1, cnt)


def _tc_mid_body(agg_ref, g_ref, cnt_ref, b_ref, w_ref, o_ref):
    dinv = _dinv_block(cnt_ref[...])
    w = w_ref[...]
    h0 = jax.nn.relu((agg_ref[0] + g_ref[0]) * dinv + b_ref[0:1, :128])
    h1 = jax.nn.relu((agg_ref[1] + g_ref[1]) * dinv + b_ref[0:1, 128:])
    h2 = (jnp.dot(h0, w[:128, :], preferred_element_type=jnp.float32)
          + jnp.dot(h1, w[128:, :], preferred_element_type=jnp.float32))
    o_ref[...] = h2 * dinv


def _tc_mid(agg1, g1, cnt, b1, W2):
    return pl.pallas_call(
        _tc_mid_body,
        grid=(_NR,),
        in_specs=[
            pl.BlockSpec((2, _R, 128), lambda i: (0, i, 0)),
            pl.BlockSpec((2, _R, 128), lambda i: (0, i, 0)),
            pl.BlockSpec((2, _R, 128), lambda i: (0, i, 0)),
            pl.BlockSpec((1, 256), lambda i: (0, 0)),
            pl.BlockSpec((256, 128), lambda i: (0, 0)),
        ],
        out_specs=pl.BlockSpec((_R, 128), lambda i: (i, 0)),
        out_shape=jax.ShapeDtypeStruct((N_NODES, 128), jnp.float32),
    )(agg1, g1, cnt, b1, W2)


def _tc_post_body(agg_ref, g_ref, cnt_ref, b_ref, o_ref):
    dinv = _dinv_block(cnt_ref[...])
    o_ref[...] = (agg_ref[0] + agg_ref[1] + g_ref[...]) * dinv + b_ref[0:1, :]


def _tc_post(agg2, g2, cnt, b2):
    return pl.pallas_call(
        _tc_post_body,
        grid=(_NR,),
        in_specs=[
            pl.BlockSpec((2, _R, 128), lambda i: (0, i, 0)),
            pl.BlockSpec((_R, 128), lambda i: (i, 0)),
            pl.BlockSpec((2, _R, 128), lambda i: (0, i, 0)),
            pl.BlockSpec((1, 128), lambda i: (0, 0)),
        ],
        out_specs=pl.BlockSpec((_R, 128), lambda i: (i, 0)),
        out_shape=jax.ShapeDtypeStruct((N_NODES, 128), jnp.float32),
    )(agg2, g2, cnt, b2)


def kernel(x, edge_index, W1, b1, W2, b2):
    src = edge_index[0].astype(jnp.int32)
    dst = edge_index[1].astype(jnp.int32)
    pad = E_PAD - N_EDGES
    src_p = jnp.concatenate([src, jnp.zeros((pad,), jnp.int32)])
    dst_p = jnp.concatenate([dst, jnp.full((pad,), N_NODES, jnp.int32)])

    src2 = jnp.stack([src_p, src_p + N_NODES]).reshape(2, 16, NCH_AGG, K)
    src2 = jnp.concatenate(
        [src2, jnp.zeros((2, 16, _PADC, K), jnp.int32)], axis=2)
    dst_agg = dst_p.reshape(16, NCH_AGG, K)
    dst_cnt = dst_p.reshape(32, NCH_CNT, K)

    src_cnt = src_p.reshape(32, NCH_CNT, K)
    src_cnt = jnp.concatenate(
        [src_cnt, jnp.zeros((32, _PADC, K), jnp.int32)], axis=1)

    ones128 = jnp.ones((K, 128), jnp.float32)
    zeros128 = jnp.zeros((R_PAD, 128), jnp.float32)

    cnt = _sc_count(dst_cnt, ones128, zeros128)[:, :N_NODES, :]

    g1 = _tc_pre(x, W1, cnt)
    agg1 = _sc_agg128(src2, dst_agg, g1.reshape(2 * N_NODES, 128), zeros128)
    g2 = _tc_mid(agg1[:, :N_NODES], g1, cnt, b1.reshape(1, 256), W2)
    agg2 = _sc_agg_esplit(src_cnt, dst_cnt, g2, zeros128)
    z = _tc_post(agg2[:, :N_NODES], g2, cnt, b2.reshape(1, 128))
    return z


# speedup vs baseline: 1.7700x; 1.7700x over previous
import functools
import jax
import jax.numpy as jnp
from jax import lax
from jax.experimental import pallas as pl
from jax.experimental.pallas import tpu as pltpu
from jax.experimental.pallas import tpu_sc as plsc

N_NODES = 10000
N_EDGES = 160000
R_PAD = 10240
E_PAD = 163840
K = 128
NCH_AGG = 80
NCH_CNT = 40
ROWS_PER_TILE = R_PAD // 16

_mesh = plsc.VectorSubcoreMesh(core_axis_name="c", subcore_axis_name="s")


@functools.partial(
    pl.kernel,
    out_type=jax.ShapeDtypeStruct((2, R_PAD, 128), jnp.float32),
    mesh=_mesh,
    scratch_types=[
        pltpu.VMEM((NCH_CNT, K), jnp.int32),
        pltpu.VMEM((K, 128), jnp.float32),
        pltpu.VMEM_SHARED((R_PAD, 128), jnp.float32),
    ],
)
def _sc_count(dst_hbm, ones_hbm, zeros_hbm, out_hbm, dst_v, ones_v, acc):
    cid = lax.axis_index("c")
    sid = lax.axis_index("s")
    wid = sid * 2 + cid
    rows = pl.ds(sid * ROWS_PER_TILE, ROWS_PER_TILE)

    pltpu.sync_copy(zeros_hbm.at[rows], acc.at[rows])
    pltpu.sync_copy(ones_hbm, ones_v)
    pltpu.sync_copy(dst_hbm.at[wid], dst_v)
    plsc.subcore_barrier()

    def body(c, carry):
        pltpu.sync_copy(ones_v, acc.at[dst_v.at[c]], add=True)
        return carry

    lax.fori_loop(0, NCH_CNT, body, 0)
    plsc.subcore_barrier()
    pltpu.sync_copy(acc.at[rows], out_hbm.at[cid, rows])


_NB = 2
_PADC = 8


def _agg_loop(nch, table_hbm, zeros_hbm, src_v, dst_v, bufs, sems, acc):
    def body(i, carry):
        c0 = i * _NB
        h0 = pltpu.async_copy(table_hbm.at[src_v.at[c0]], bufs[0], sems[0])
        h1 = pltpu.async_copy(table_hbm.at[src_v.at[c0 + 1]], bufs[1], sems[1])
        h0.wait()
        pltpu.sync_copy(bufs[0], acc.at[dst_v.at[c0]], add=True)
        h1.wait()
        pltpu.sync_copy(bufs[1], acc.at[dst_v.at[c0 + 1]], add=True)
        return carry

    lax.fori_loop(0, nch // _NB, body, 0)


_PH = 2
_CH = NCH_AGG // _PH


@functools.partial(
    pl.kernel,
    out_type=jax.ShapeDtypeStruct((2, R_PAD, 128), jnp.float32),
    mesh=_mesh,
    scratch_types=[
        pltpu.VMEM((_CH + _PADC, K), jnp.int32),
        pltpu.VMEM((_CH, K), jnp.int32),
        pltpu.VMEM((K, 128), jnp.float32),
        pltpu.VMEM((K, 128), jnp.float32),
        pltpu.VMEM_SHARED((R_PAD, 128), jnp.float32),
        pltpu.SemaphoreType.DMA,
        pltpu.SemaphoreType.DMA,
    ],
)
def _sc_agg128(src2_hbm, dst_hbm, table_hbm, zeros_hbm, out_hbm,
               src_v, dst_v, buf0, buf1, acc, sem0, sem1):
    cid = lax.axis_index("c")
    sid = lax.axis_index("s")
    rows = pl.ds(sid * ROWS_PER_TILE, ROWS_PER_TILE)

    pltpu.sync_copy(zeros_hbm.at[rows], acc.at[rows])
    plsc.subcore_barrier()

    def phase(p, carry):
        pltpu.sync_copy(src2_hbm.at[cid, sid, pl.ds(p * _CH, _CH + _PADC)],
                        src_v)
        pltpu.sync_copy(dst_hbm.at[sid, pl.ds(p * _CH, _CH)], dst_v)
        _agg_loop(_CH, table_hbm, zeros_hbm, src_v, dst_v,
                  (buf0, buf1), (sem0, sem1), acc)
        return carry

    lax.fori_loop(0, _PH, phase, 0)
    plsc.subcore_barrier()
    pltpu.sync_copy(acc.at[rows], out_hbm.at[cid, rows])


@functools.partial(
    pl.kernel,
    out_type=jax.ShapeDtypeStruct((2, R_PAD, 128), jnp.float32),
    mesh=_mesh,
    scratch_types=[
        pltpu.VMEM((NCH_CNT + _PADC, K), jnp.int32),
        pltpu.VMEM((NCH_CNT, K), jnp.int32),
        pltpu.VMEM((K, 128), jnp.float32),
        pltpu.VMEM((K, 128), jnp.float32),
        pltpu.VMEM_SHARED((R_PAD, 128), jnp.float32),
        pltpu.SemaphoreType.DMA,
        pltpu.SemaphoreType.DMA,
    ],
)
def _sc_agg_esplit(src_hbm, dst_hbm, table_hbm, zeros_hbm, out_hbm,
                   src_v, dst_v, buf0, buf1, acc, sem0, sem1):
    cid = lax.axis_index("c")
    sid = lax.axis_index("s")
    wid = sid * 2 + cid
    rows = pl.ds(sid * ROWS_PER_TILE, ROWS_PER_TILE)

    pltpu.sync_copy(zeros_hbm.at[rows], acc.at[rows])
    pltpu.sync_copy(src_hbm.at[wid], src_v)
    pltpu.sync_copy(dst_hbm.at[wid], dst_v)
    plsc.subcore_barrier()
    _agg_loop(NCH_CNT, table_hbm, zeros_hbm, src_v, dst_v,
              (buf0, buf1), (sem0, sem1), acc)
    plsc.subcore_barrier()
    pltpu.sync_copy(acc.at[rows], out_hbm.at[cid, rows])


_R = 2000
_NR = N_NODES // _R


def _dinv_block(cnt_blk):
    deg = cnt_blk[0] + cnt_blk[1] + 1.0
    return lax.rsqrt(deg)[:, 0:1]


def _tc_pre_body(x_ref, w_ref, cnt_ref, o_ref):
    dinv = _dinv_block(cnt_ref[...])
    h = jnp.dot(x_ref[...], w_ref[...], preferred_element_type=jnp.float32)
    g = h * dinv
    o_ref[0] = g[:, :128]
    o_ref[1] = g[:, 128:]


def _tc_pre(x, W1, cnt):
    return pl.pallas_call(
        _tc_pre_body,
        grid=(_NR,),
        in_specs=[
            pl.BlockSpec((_R, 256), lambda i: (i, 0)),
            pl.BlockSpec((256, 256), lambda i: (0, 0)),
            pl.BlockSpec((2, _R, 128), lambda i: (0, i, 0)),
        ],
        out_specs=pl.BlockSpec((2, _R, 128), lambda i: (0, i, 0)),
        out_shape=jax.ShapeDtypeStruct((2, N_NODES, 128), jnp.float32),
    )(x, W1, cnt)


def _tc_mid_body(agg_ref, g_ref, cnt_ref, b_ref, w_ref, o_ref):
    dinv = _dinv_block(cnt_ref[...])
    w = w_ref[...]
    h0 = jax.nn.relu((agg_ref[0] + g_ref[0]) * dinv + b_ref[0:1, :128])
    h1 = jax.nn.relu((agg_ref[1] + g_ref[1]) * dinv + b_ref[0:1, 128:])
    h2 = (jnp.dot(h0, w[:128, :], preferred_element_type=jnp.float32)
          + jnp.dot(h1, w[128:, :], preferred_element_type=jnp.float32))
    o_ref[...] = h2 * dinv


def _tc_mid(agg1, g1, cnt, b1, W2):
    return pl.pallas_call(
        _tc_mid_body,
        grid=(_NR,),
        in_specs=[
            pl.BlockSpec((2, _R, 128), lambda i: (0, i, 0)),
            pl.BlockSpec((2, _R, 128), lambda i: (0, i, 0)),
            pl.BlockSpec((2, _R, 128), lambda i: (0, i, 0)),
            pl.BlockSpec((1, 256), lambda i: (0, 0)),
            pl.BlockSpec((256, 128), lambda i: (0, 0)),
        ],
        out_specs=pl.BlockSpec((_R, 128), lambda i: (i, 0)),
        out_shape=jax.ShapeDtypeStruct((N_NODES, 128), jnp.float32),
    )(agg1, g1, cnt, b1, W2)


def _tc_post_body(agg_ref, g_ref, cnt_ref, b_ref, o_ref):
    dinv = _dinv_block(cnt_ref[...])
    o_ref[...] = (agg_ref[0] + agg_ref[1] + g_ref[...]) * dinv + b_ref[0:1, :]


def _tc_post(agg2, g2, cnt, b2):
    return pl.pallas_call(
        _tc_post_body,
        grid=(_NR,),
        in_specs=[
            pl.BlockSpec((2, _R, 128), lambda i: (0, i, 0)),
            pl.BlockSpec((_R, 128), lambda i: (i, 0)),
            pl.BlockSpec((2, _R, 128), lambda i: (0, i, 0)),
            pl.BlockSpec((1, 128), lambda i: (0, 0)),
        ],
        out_specs=pl.BlockSpec((_R, 128), lambda i: (i, 0)),
        out_shape=jax.ShapeDtypeStruct((N_NODES, 128), jnp.float32),
    )(agg2, g2, cnt, b2)


def kernel(x, edge_index, W1, b1, W2, b2):
    src = edge_index[0].astype(jnp.int32)
    dst = edge_index[1].astype(jnp.int32)
    pad = E_PAD - N_EDGES
    src_p = jnp.concatenate([src, jnp.zeros((pad,), jnp.int32)])
    dst_p = jnp.concatenate([dst, jnp.full((pad,), N_NODES, jnp.int32)])

    src2 = jnp.stack([src_p, src_p + N_NODES]).reshape(2, 16, NCH_AGG, K)
    src2 = jnp.concatenate(
        [src2, jnp.zeros((2, 16, _PADC, K), jnp.int32)], axis=2)
    dst_agg = dst_p.reshape(16, NCH_AGG, K)
    dst_cnt = dst_p.reshape(32, NCH_CNT, K)

    src_cnt = src_p.reshape(32, NCH_CNT, K)
    src_cnt = jnp.concatenate(
        [src_cnt, jnp.zeros((32, _PADC, K), jnp.int32)], axis=1)

    ones128 = jnp.ones((K, 128), jnp.float32)
    zeros128 = jnp.zeros((R_PAD, 128), jnp.float32)

    cnt = _sc_count(dst_cnt, ones128, zeros128)[:, :N_NODES, :]

    g1 = _tc_pre(x, W1, cnt)
    agg1 = _sc_agg128(src2, dst_agg, g1.reshape(2 * N_NODES, 128), zeros128)
    g2 = _tc_mid(agg1[:, :N_NODES], g1, cnt, b1.reshape(1, 256), W2)
    agg2 = _sc_agg_esplit(src_cnt, dst_cnt, g2, zeros128)
    z = _tc_post(agg2[:, :N_NODES], g2, cnt, b2.reshape(1, 128))
    return z


# speedup vs baseline: 1.9150x; 1.0820x over previous
import functools
import jax
import jax.numpy as jnp
from jax import lax
from jax.experimental import pallas as pl
from jax.experimental.pallas import tpu as pltpu
from jax.experimental.pallas import tpu_sc as plsc

N_NODES = 10000
N_EDGES = 160000
R_PAD = 10240
E_PAD = 163840
K = 128
NCH_AGG = 80
NCH_CNT = 40
ROWS_PER_TILE = R_PAD // 16
_PADC = 8
_UNROLL = 8

_mesh = plsc.VectorSubcoreMesh(core_axis_name="c", subcore_axis_name="s")


@functools.partial(
    pl.kernel,
    out_type=jax.ShapeDtypeStruct((2, R_PAD, 128), jnp.float32),
    mesh=_mesh,
    scratch_types=[
        pltpu.VMEM((NCH_CNT, K), jnp.int32),
        pltpu.VMEM((K, 128), jnp.float32),
        pltpu.VMEM_SHARED((R_PAD, 128), jnp.float32),
    ],
)
def _sc_count(dst_hbm, ones_hbm, zeros_hbm, out_hbm, dst_v, ones_v, acc):
    cid = lax.axis_index("c")
    sid = lax.axis_index("s")
    wid = sid * 2 + cid
    rows = pl.ds(sid * ROWS_PER_TILE, ROWS_PER_TILE)

    pltpu.sync_copy(zeros_hbm.at[rows], acc.at[rows])
    pltpu.sync_copy(ones_hbm, ones_v)
    pltpu.sync_copy(dst_hbm.at[wid], dst_v)
    plsc.subcore_barrier()

    def body(c, carry):
        pltpu.sync_copy(ones_v, acc.at[dst_v.at[c]], add=True)
        return carry

    lax.fori_loop(0, NCH_CNT, body, 0)
    plsc.subcore_barrier()
    pltpu.sync_copy(acc.at[rows], out_hbm.at[cid, rows])


def _agg_ring(nseg, table_hbm, src_v, dst_v, bufs, sems, acc):
    def body(i, carry):
        c0 = i * _UNROLL
        hs = [
            pltpu.async_copy(table_hbm.at[src_v.at[c0]], bufs[0], sems[0]),
            pltpu.async_copy(table_hbm.at[src_v.at[c0 + 1]], bufs[1], sems[1]),
        ]
        for j in range(_UNROLL):
            b = j & 1
            hs[b].wait()
            pltpu.sync_copy(bufs[b], acc.at[dst_v.at[c0 + j]], add=True)
            if j < _UNROLL - 2:
                hs[b] = pltpu.async_copy(
                    table_hbm.at[src_v.at[c0 + j + 2]], bufs[b], sems[b])
        return carry

    lax.fori_loop(0, nseg, body, 0)


_PH = 2
_CH = NCH_AGG // _PH


@functools.partial(
    pl.kernel,
    out_type=jax.ShapeDtypeStruct((2, R_PAD, 128), jnp.float32),
    mesh=_mesh,
    scratch_types=[
        pltpu.VMEM((_CH + _PADC, K), jnp.int32),
        pltpu.VMEM((_CH, K), jnp.int32),
        pltpu.VMEM((K, 128), jnp.float32),
        pltpu.VMEM((K, 128), jnp.float32),
        pltpu.VMEM_SHARED((R_PAD, 128), jnp.float32),
        pltpu.SemaphoreType.DMA,
        pltpu.SemaphoreType.DMA,
    ],
)
def _sc_agg128(src2_hbm, dst_hbm, table_hbm, zeros_hbm, out_hbm,
               src_v, dst_v, buf0, buf1, acc, sem0, sem1):
    cid = lax.axis_index("c")
    sid = lax.axis_index("s")
    rows = pl.ds(sid * ROWS_PER_TILE, ROWS_PER_TILE)

    pltpu.sync_copy(zeros_hbm.at[rows], acc.at[rows])
    plsc.subcore_barrier()

    def phase(p, carry):
        pltpu.sync_copy(src2_hbm.at[cid, sid, pl.ds(p * _CH, _CH + _PADC)],
                        src_v)
        pltpu.sync_copy(dst_hbm.at[sid, pl.ds(p * _CH, _CH)], dst_v)
        _agg_ring(_CH // _UNROLL, table_hbm, src_v, dst_v,
                  (buf0, buf1), (sem0, sem1), acc)
        return carry

    lax.fori_loop(0, _PH, phase, 0)
    plsc.subcore_barrier()
    pltpu.sync_copy(acc.at[rows], out_hbm.at[cid, rows])


@functools.partial(
    pl.kernel,
    out_type=jax.ShapeDtypeStruct((2, R_PAD, 128), jnp.float32),
    mesh=_mesh,
    scratch_types=[
        pltpu.VMEM((NCH_CNT + _PADC, K), jnp.int32),
        pltpu.VMEM((NCH_CNT, K), jnp.int32),
        pltpu.VMEM((K, 128), jnp.float32),
        pltpu.VMEM((K, 128), jnp.float32),
        pltpu.VMEM_SHARED((R_PAD, 128), jnp.float32),
        pltpu.SemaphoreType.DMA,
        pltpu.SemaphoreType.DMA,
    ],
)
def _sc_agg_esplit(src_hbm, dst_hbm, table_hbm, zeros_hbm, out_hbm,
                   src_v, dst_v, buf0, buf1, acc, sem0, sem1):
    cid = lax.axis_index("c")
    sid = lax.axis_index("s")
    wid = sid * 2 + cid
    rows = pl.ds(sid * ROWS_PER_TILE, ROWS_PER_TILE)

    pltpu.sync_copy(zeros_hbm.at[rows], acc.at[rows])
    pltpu.sync_copy(src_hbm.at[wid], src_v)
    pltpu.sync_copy(dst_hbm.at[wid], dst_v)
    plsc.subcore_barrier()
    _agg_ring(NCH_CNT // _UNROLL, table_hbm, src_v, dst_v,
              (buf0, buf1), (sem0, sem1), acc)
    plsc.subcore_barrier()
    pltpu.sync_copy(acc.at[rows], out_hbm.at[cid, rows])


_R = 2000
_NR = N_NODES // _R


def _dinv_block(cnt_blk):
    deg = cnt_blk[0] + cnt_blk[1] + 1.0
    return lax.rsqrt(deg)[:, 0:1]


def _tc_pre_body(x_ref, w_ref, cnt_ref, o_ref):
    dinv = _dinv_block(cnt_ref[...])
    h = jnp.dot(x_ref[...], w_ref[...], preferred_element_type=jnp.float32)
    g = h * dinv
    o_ref[0] = g[:, :128]
    o_ref[1] = g[:, 128:]


def _tc_pre(x, W1, cnt):
    return pl.pallas_call(
        _tc_pre_body,
        grid=(_NR,),
        in_specs=[
            pl.BlockSpec((_R, 256), lambda i: (i, 0)),
            pl.BlockSpec((256, 256), lambda i: (0, 0)),
            pl.BlockSpec((2, _R, 128), lambda i: (0, i, 0)),
        ],
        out_specs=pl.BlockSpec((2, _R, 128), lambda i: (0, i, 0)),
        out_shape=jax.ShapeDtypeStruct((2, N_NODES, 128), jnp.float32),
    )(x, W1, cnt)


def _tc_mid_body(agg_ref, g_ref, cnt_ref, b_ref, w_ref, o_ref):
    dinv = _dinv_block(cnt_ref[...])
    w = w_ref[...]
    h0 = jax.nn.relu((agg_ref[0] + g_ref[0]) * dinv + b_ref[0:1, :128])
    h1 = jax.nn.relu((agg_ref[1] + g_ref[1]) * dinv + b_ref[0:1, 128:])
    h2 = (jnp.dot(h0, w[:128, :], preferred_element_type=jnp.float32)
          + jnp.dot(h1, w[128:, :], preferred_element_type=jnp.float32))
    o_ref[...] = h2 * dinv


def _tc_mid(agg1, g1, cnt, b1, W2):
    return pl.pallas_call(
        _tc_mid_body,
        grid=(_NR,),
        in_specs=[
            pl.BlockSpec((2, _R, 128), lambda i: (0, i, 0)),
            pl.BlockSpec((2, _R, 128), lambda i: (0, i, 0)),
            pl.BlockSpec((2, _R, 128), lambda i: (0, i, 0)),
            pl.BlockSpec((1, 256), lambda i: (0, 0)),
            pl.BlockSpec((256, 128), lambda i: (0, 0)),
        ],
        out_specs=pl.BlockSpec((_R, 128), lambda i: (i, 0)),
        out_shape=jax.ShapeDtypeStruct((N_NODES, 128), jnp.float32),
    )(agg1, g1, cnt, b1, W2)


def _tc_post_body(agg_ref, g_ref, cnt_ref, b_ref, o_ref):
    dinv = _dinv_block(cnt_ref[...])
    o_ref[...] = (agg_ref[0] + agg_ref[1] + g_ref[...]) * dinv + b_ref[0:1, :]


def _tc_post(agg2, g2, cnt, b2):
    return pl.pallas_call(
        _tc_post_body,
        grid=(_NR,),
        in_specs=[
            pl.BlockSpec((2, _R, 128), lambda i: (0, i, 0)),
            pl.BlockSpec((_R, 128), lambda i: (i, 0)),
            pl.BlockSpec((2, _R, 128), lambda i: (0, i, 0)),
            pl.BlockSpec((1, 128), lambda i: (0, 0)),
        ],
        out_specs=pl.BlockSpec((_R, 128), lambda i: (i, 0)),
        out_shape=jax.ShapeDtypeStruct((N_NODES, 128), jnp.float32),
    )(agg2, g2, cnt, b2)


def kernel(x, edge_index, W1, b1, W2, b2):
    src = edge_index[0].astype(jnp.int32)
    dst = edge_index[1].astype(jnp.int32)
    pad = E_PAD - N_EDGES
    src_p = jnp.concatenate([src, jnp.zeros((pad,), jnp.int32)])
    dst_p = jnp.concatenate([dst, jnp.full((pad,), N_NODES, jnp.int32)])

    src2 = jnp.stack([src_p, src_p + N_NODES]).reshape(2, 16, NCH_AGG, K)
    src2 = jnp.concatenate(
        [src2, jnp.zeros((2, 16, _PADC, K), jnp.int32)], axis=2)
    dst_agg = dst_p.reshape(16, NCH_AGG, K)
    dst_cnt = dst_p.reshape(32, NCH_CNT, K)

    src_cnt = src_p.reshape(32, NCH_CNT, K)
    src_cnt = jnp.concatenate(
        [src_cnt, jnp.zeros((32, _PADC, K), jnp.int32)], axis=1)

    ones128 = jnp.ones((K, 128), jnp.float32)
    zeros128 = jnp.zeros((R_PAD, 128), jnp.float32)

    cnt = _sc_count(dst_cnt, ones128, zeros128)[:, :N_NODES, :]

    g1 = _tc_pre(x, W1, cnt)
    agg1 = _sc_agg128(src2, dst_agg, g1.reshape(2 * N_NODES, 128), zeros128)
    g2 = _tc_mid(agg1[:, :N_NODES], g1, cnt, b1.reshape(1, 256), W2)
    agg2 = _sc_agg_esplit(src_cnt, dst_cnt, g2, zeros128)
    z = _tc_post(agg2[:, :N_NODES], g2, cnt, b2.reshape(1, 128))
    return z


# speedup vs baseline: 1.9280x; 1.0068x over previous
import functools
import jax
import jax.numpy as jnp
from jax import lax
from jax.experimental import pallas as pl
from jax.experimental.pallas import tpu as pltpu
from jax.experimental.pallas import tpu_sc as plsc

N_NODES = 10000
N_EDGES = 160000
R_PAD = 10240
E_PAD = 163840
K = 128
NCH_AGG = 80
NCH_CNT = 40
ROWS_PER_TILE = R_PAD // 16
_PADC = 8
_UNROLL = 10

_mesh = plsc.VectorSubcoreMesh(core_axis_name="c", subcore_axis_name="s")


@functools.partial(
    pl.kernel,
    out_type=jax.ShapeDtypeStruct((2, R_PAD, 128), jnp.float32),
    mesh=_mesh,
    scratch_types=[
        pltpu.VMEM((NCH_CNT, K), jnp.int32),
        pltpu.VMEM((K, 128), jnp.float32),
        pltpu.VMEM_SHARED((R_PAD, 128), jnp.float32),
    ],
)
def _sc_count(dst_hbm, ones_hbm, zeros_hbm, out_hbm, dst_v, ones_v, acc):
    cid = lax.axis_index("c")
    sid = lax.axis_index("s")
    wid = sid * 2 + cid
    rows = pl.ds(sid * ROWS_PER_TILE, ROWS_PER_TILE)

    pltpu.sync_copy(zeros_hbm.at[pl.ds(0, K)], ones_v)
    for t in range(ROWS_PER_TILE // K):
        pltpu.sync_copy(ones_v, acc.at[pl.ds(sid * ROWS_PER_TILE + t * K, K)])
    pltpu.sync_copy(ones_hbm, ones_v)
    pltpu.sync_copy(dst_hbm.at[wid], dst_v)
    plsc.subcore_barrier()

    def body(c, carry):
        pltpu.sync_copy(ones_v, acc.at[dst_v.at[c]], add=True)
        return carry

    lax.fori_loop(0, NCH_CNT, body, 0)
    plsc.subcore_barrier()
    pltpu.sync_copy(acc.at[rows], out_hbm.at[cid, rows])


def _agg_ring(nseg, table_hbm, src_v, dst_v, bufs, sems, acc):
    def body(i, carry):
        c0 = i * _UNROLL
        hs = [
            pltpu.async_copy(table_hbm.at[src_v.at[c0]], bufs[0], sems[0]),
            pltpu.async_copy(table_hbm.at[src_v.at[c0 + 1]], bufs[1], sems[1]),
        ]
        for j in range(_UNROLL):
            b = j & 1
            hs[b].wait()
            pltpu.sync_copy(bufs[b], acc.at[dst_v.at[c0 + j]], add=True)
            if j < _UNROLL - 2:
                hs[b] = pltpu.async_copy(
                    table_hbm.at[src_v.at[c0 + j + 2]], bufs[b], sems[b])
        return carry

    lax.fori_loop(0, nseg, body, 0)


_PH = 2
_CH = NCH_AGG // _PH


@functools.partial(
    pl.kernel,
    out_type=jax.ShapeDtypeStruct((2, R_PAD, 128), jnp.float32),
    mesh=_mesh,
    scratch_types=[
        pltpu.VMEM((_CH + _PADC, K), jnp.int32),
        pltpu.VMEM((_CH, K), jnp.int32),
        pltpu.VMEM((K, 128), jnp.float32),
        pltpu.VMEM((K, 128), jnp.float32),
        pltpu.VMEM_SHARED((R_PAD, 128), jnp.float32),
        pltpu.SemaphoreType.DMA,
        pltpu.SemaphoreType.DMA,
    ],
)
def _sc_agg128(src2_hbm, dst_hbm, table_hbm, zeros_hbm, out_hbm,
               src_v, dst_v, buf0, buf1, acc, sem0, sem1):
    cid = lax.axis_index("c")
    sid = lax.axis_index("s")
    rows = pl.ds(sid * ROWS_PER_TILE, ROWS_PER_TILE)

    pltpu.sync_copy(zeros_hbm.at[pl.ds(0, K)], buf0)
    for t in range(ROWS_PER_TILE // K):
        pltpu.sync_copy(buf0, acc.at[pl.ds(sid * ROWS_PER_TILE + t * K, K)])
    plsc.subcore_barrier()

    def phase(p, carry):
        pltpu.sync_copy(src2_hbm.at[cid, sid, pl.ds(p * _CH, _CH + _PADC)],
                        src_v)
        pltpu.sync_copy(dst_hbm.at[sid, pl.ds(p * _CH, _CH)], dst_v)
        _agg_ring(_CH // _UNROLL, table_hbm, src_v, dst_v,
                  (buf0, buf1), (sem0, sem1), acc)
        return carry

    lax.fori_loop(0, _PH, phase, 0)
    plsc.subcore_barrier()
    pltpu.sync_copy(acc.at[rows], out_hbm.at[cid, rows])


@functools.partial(
    pl.kernel,
    out_type=jax.ShapeDtypeStruct((2, R_PAD, 128), jnp.float32),
    mesh=_mesh,
    scratch_types=[
        pltpu.VMEM((NCH_CNT + _PADC, K), jnp.int32),
        pltpu.VMEM((NCH_CNT, K), jnp.int32),
        pltpu.VMEM((K, 128), jnp.float32),
        pltpu.VMEM((K, 128), jnp.float32),
        pltpu.VMEM_SHARED((R_PAD, 128), jnp.float32),
        pltpu.SemaphoreType.DMA,
        pltpu.SemaphoreType.DMA,
    ],
)
def _sc_agg_esplit(src_hbm, dst_hbm, table_hbm, zeros_hbm, out_hbm,
                   src_v, dst_v, buf0, buf1, acc, sem0, sem1):
    cid = lax.axis_index("c")
    sid = lax.axis_index("s")
    wid = sid * 2 + cid
    rows = pl.ds(sid * ROWS_PER_TILE, ROWS_PER_TILE)

    pltpu.sync_copy(zeros_hbm.at[pl.ds(0, K)], buf0)
    for t in range(ROWS_PER_TILE // K):
        pltpu.sync_copy(buf0, acc.at[pl.ds(sid * ROWS_PER_TILE + t * K, K)])
    pltpu.sync_copy(src_hbm.at[wid], src_v)
    pltpu.sync_copy(dst_hbm.at[wid], dst_v)
    plsc.subcore_barrier()
    _agg_ring(NCH_CNT // _UNROLL, table_hbm, src_v, dst_v,
              (buf0, buf1), (sem0, sem1), acc)
    plsc.subcore_barrier()
    pltpu.sync_copy(acc.at[rows], out_hbm.at[cid, rows])


_R = 2000
_NR = N_NODES // _R


def _dinv_block(cnt_blk):
    deg = cnt_blk[0] + cnt_blk[1] + 1.0
    return lax.rsqrt(deg)[:, 0:1]


def _tc_pre_body(x_ref, w_ref, cnt_ref, o_ref):
    dinv = _dinv_block(cnt_ref[...])
    h = jnp.dot(x_ref[...], w_ref[...], preferred_element_type=jnp.float32)
    g = h * dinv
    o_ref[0] = g[:, :128]
    o_ref[1] = g[:, 128:]


def _tc_pre(x, W1, cnt):
    return pl.pallas_call(
        _tc_pre_body,
        grid=(_NR,),
        in_specs=[
            pl.BlockSpec((_R, 256), lambda i: (i, 0)),
            pl.BlockSpec((256, 256), lambda i: (0, 0)),
            pl.BlockSpec((2, _R, 128), lambda i: (0, i, 0)),
        ],
        out_specs=pl.BlockSpec((2, _R, 128), lambda i: (0, i, 0)),
        out_shape=jax.ShapeDtypeStruct((2, N_NODES, 128), jnp.float32),
    )(x, W1, cnt)


def _tc_mid_body(agg_ref, g_ref, cnt_ref, b_ref, w_ref, o_ref):
    dinv = _dinv_block(cnt_ref[...])
    w = w_ref[...]
    h0 = jax.nn.relu((agg_ref[0] + g_ref[0]) * dinv + b_ref[0:1, :128])
    h1 = jax.nn.relu((agg_ref[1] + g_ref[1]) * dinv + b_ref[0:1, 128:])
    h2 = (jnp.dot(h0, w[:128, :], preferred_element_type=jnp.float32)
          + jnp.dot(h1, w[128:, :], preferred_element_type=jnp.float32))
    o_ref[...] = h2 * dinv


def _tc_mid(agg1, g1, cnt, b1, W2):
    return pl.pallas_call(
        _tc_mid_body,
        grid=(_NR,),
        in_specs=[
            pl.BlockSpec((2, _R, 128), lambda i: (0, i, 0)),
            pl.BlockSpec((2, _R, 128), lambda i: (0, i, 0)),
            pl.BlockSpec((2, _R, 128), lambda i: (0, i, 0)),
            pl.BlockSpec((1, 256), lambda i: (0, 0)),
            pl.BlockSpec((256, 128), lambda i: (0, 0)),
        ],
        out_specs=pl.BlockSpec((_R, 128), lambda i: (i, 0)),
        out_shape=jax.ShapeDtypeStruct((N_NODES, 128), jnp.float32),
    )(agg1, g1, cnt, b1, W2)


def _tc_post_body(agg_ref, g_ref, cnt_ref, b_ref, o_ref):
    dinv = _dinv_block(cnt_ref[...])
    o_ref[...] = (agg_ref[0] + agg_ref[1] + g_ref[...]) * dinv + b_ref[0:1, :]


def _tc_post(agg2, g2, cnt, b2):
    return pl.pallas_call(
        _tc_post_body,
        grid=(_NR,),
        in_specs=[
            pl.BlockSpec((2, _R, 128), lambda i: (0, i, 0)),
            pl.BlockSpec((_R, 128), lambda i: (i, 0)),
            pl.BlockSpec((2, _R, 128), lambda i: (0, i, 0)),
            pl.BlockSpec((1, 128), lambda i: (0, 0)),
        ],
        out_specs=pl.BlockSpec((_R, 128), lambda i: (i, 0)),
        out_shape=jax.ShapeDtypeStruct((N_NODES, 128), jnp.float32),
    )(agg2, g2, cnt, b2)


def kernel(x, edge_index, W1, b1, W2, b2):
    src = edge_index[0].astype(jnp.int32)
    dst = edge_index[1].astype(jnp.int32)
    pad = E_PAD - N_EDGES
    src_p = jnp.concatenate([src, jnp.zeros((pad,), jnp.int32)])
    dst_p = jnp.concatenate([dst, jnp.full((pad,), N_NODES, jnp.int32)])

    src2 = jnp.stack([src_p, src_p + N_NODES]).reshape(2, 16, NCH_AGG, K)
    src2 = jnp.concatenate(
        [src2, jnp.zeros((2, 16, _PADC, K), jnp.int32)], axis=2)
    dst_agg = dst_p.reshape(16, NCH_AGG, K)
    dst_cnt = dst_p.reshape(32, NCH_CNT, K)

    src_cnt = src_p.reshape(32, NCH_CNT, K)
    src_cnt = jnp.concatenate(
        [src_cnt, jnp.zeros((32, _PADC, K), jnp.int32)], axis=1)

    ones128 = jnp.ones((K, 128), jnp.float32)
    zeros128 = jnp.zeros((R_PAD, 128), jnp.float32)

    cnt = _sc_count(dst_cnt, ones128, zeros128)[:, :N_NODES, :]

    g1 = _tc_pre(x, W1, cnt)
    agg1 = _sc_agg128(src2, dst_agg, g1.reshape(2 * N_NODES, 128), zeros128)
    g2 = _tc_mid(agg1[:, :N_NODES], g1, cnt, b1.reshape(1, 256), W2)
    agg2 = _sc_agg_esplit(src_cnt, dst_cnt, g2, zeros128)
    z = _tc_post(agg2[:, :N_NODES], g2, cnt, b2.reshape(1, 128))
    return z


# speedup vs baseline: 1.9529x; 1.0129x over previous
import functools
import jax
import jax.numpy as jnp
from jax import lax
from jax.experimental import pallas as pl
from jax.experimental.pallas import tpu as pltpu
from jax.experimental.pallas import tpu_sc as plsc

N_NODES = 10000
N_EDGES = 160000
R_PAD = 10240
E_PAD = 163840
K = 128
NCH_AGG = 80
NCH_CNT = 40
ROWS_PER_TILE = R_PAD // 16
_PADC = 8
_UNROLL = 20

_mesh = plsc.VectorSubcoreMesh(core_axis_name="c", subcore_axis_name="s")


@functools.partial(
    pl.kernel,
    out_type=jax.ShapeDtypeStruct((2, R_PAD, 128), jnp.float32),
    mesh=_mesh,
    scratch_types=[
        pltpu.VMEM((NCH_CNT, K), jnp.int32),
        pltpu.VMEM((K, 128), jnp.float32),
        pltpu.VMEM_SHARED((R_PAD, 128), jnp.float32),
    ],
)
def _sc_count(dst_hbm, ones_hbm, zeros_hbm, out_hbm, dst_v, ones_v, acc):
    cid = lax.axis_index("c")
    sid = lax.axis_index("s")
    wid = sid * 2 + cid
    rows = pl.ds(sid * ROWS_PER_TILE, ROWS_PER_TILE)

    pltpu.sync_copy(zeros_hbm.at[pl.ds(0, K)], ones_v)
    for t in range(ROWS_PER_TILE // K):
        pltpu.sync_copy(ones_v, acc.at[pl.ds(sid * ROWS_PER_TILE + t * K, K)])
    pltpu.sync_copy(ones_hbm, ones_v)
    pltpu.sync_copy(dst_hbm.at[wid], dst_v)
    plsc.subcore_barrier()

    def body(c, carry):
        pltpu.sync_copy(ones_v, acc.at[dst_v.at[c]], add=True)
        return carry

    lax.fori_loop(0, NCH_CNT, body, 0)
    plsc.subcore_barrier()
    pltpu.sync_copy(acc.at[rows], out_hbm.at[cid, rows])


def _agg_ring(nseg, table_hbm, src_v, dst_v, bufs, sems, acc):
    nb = len(bufs)

    def body(i, carry):
        c0 = i * _UNROLL
        hs = [
            pltpu.async_copy(table_hbm.at[src_v.at[c0 + b]], bufs[b], sems[b])
            for b in range(nb)
        ]
        for j in range(_UNROLL):
            b = j % nb
            hs[b].wait()
            pltpu.sync_copy(bufs[b], acc.at[dst_v.at[c0 + j]], add=True)
            if j < _UNROLL - nb:
                hs[b] = pltpu.async_copy(
                    table_hbm.at[src_v.at[c0 + j + nb]], bufs[b], sems[b])
        return carry

    lax.fori_loop(0, nseg, body, 0)


_PH = 2
_CH = NCH_AGG // _PH


@functools.partial(
    pl.kernel,
    out_type=jax.ShapeDtypeStruct((2, R_PAD, 128), jnp.float32),
    mesh=_mesh,
    scratch_types=[
        pltpu.VMEM((_CH + _PADC, K), jnp.int32),
        pltpu.VMEM((_CH, K), jnp.int32),
        pltpu.VMEM((K, 128), jnp.float32),
        pltpu.VMEM((K, 128), jnp.float32),
        pltpu.VMEM_SHARED((R_PAD, 128), jnp.float32),
        pltpu.SemaphoreType.DMA,
        pltpu.SemaphoreType.DMA,
    ],
)
def _sc_agg128(src2_hbm, dst_hbm, table_hbm, zeros_hbm, out_hbm,
               src_v, dst_v, buf0, buf1, acc, sem0, sem1):
    cid = lax.axis_index("c")
    sid = lax.axis_index("s")
    rows = pl.ds(sid * ROWS_PER_TILE, ROWS_PER_TILE)

    pltpu.sync_copy(zeros_hbm.at[pl.ds(0, K)], buf0)
    for t in range(ROWS_PER_TILE // K):
        pltpu.sync_copy(buf0, acc.at[pl.ds(sid * ROWS_PER_TILE + t * K, K)])
    plsc.subcore_barrier()

    def phase(p, carry):
        pltpu.sync_copy(src2_hbm.at[cid, sid, pl.ds(p * _CH, _CH + _PADC)],
                        src_v)
        pltpu.sync_copy(dst_hbm.at[sid, pl.ds(p * _CH, _CH)], dst_v)
        _agg_ring(_CH // _UNROLL, table_hbm, src_v, dst_v,
                  (buf0, buf1), (sem0, sem1), acc)
        return carry

    lax.fori_loop(0, _PH, phase, 0)
    plsc.subcore_barrier()
    pltpu.sync_copy(acc.at[rows], out_hbm.at[cid, rows])


@functools.partial(
    pl.kernel,
    out_type=jax.ShapeDtypeStruct((2, R_PAD, 128), jnp.float32),
    mesh=_mesh,
    scratch_types=[
        pltpu.VMEM((NCH_CNT + _PADC, K), jnp.int32),
        pltpu.VMEM((NCH_CNT, K), jnp.int32),
        pltpu.VMEM((K, 128), jnp.float32),
        pltpu.VMEM((K, 128), jnp.float32),
        pltpu.VMEM_SHARED((R_PAD, 128), jnp.float32),
        pltpu.SemaphoreType.DMA,
        pltpu.SemaphoreType.DMA,
    ],
)
def _sc_agg_esplit(src_hbm, dst_hbm, table_hbm, zeros_hbm, out_hbm,
                   src_v, dst_v, buf0, buf1, acc, sem0, sem1):
    cid = lax.axis_index("c")
    sid = lax.axis_index("s")
    wid = sid * 2 + cid
    rows = pl.ds(sid * ROWS_PER_TILE, ROWS_PER_TILE)

    pltpu.sync_copy(zeros_hbm.at[pl.ds(0, K)], buf0)
    for t in range(ROWS_PER_TILE // K):
        pltpu.sync_copy(buf0, acc.at[pl.ds(sid * ROWS_PER_TILE + t * K, K)])
    pltpu.sync_copy(src_hbm.at[wid], src_v)
    pltpu.sync_copy(dst_hbm.at[wid], dst_v)
    plsc.subcore_barrier()
    _agg_ring(NCH_CNT // _UNROLL, table_hbm, src_v, dst_v,
              (buf0, buf1), (sem0, sem1), acc)
    plsc.subcore_barrier()
    pltpu.sync_copy(acc.at[rows], out_hbm.at[cid, rows])


_R = 2000
_NR = N_NODES // _R


def _dinv_block(cnt_blk):
    deg = cnt_blk[0] + cnt_blk[1] + 1.0
    return lax.rsqrt(deg)[:, 0:1]


def _tc_pre_body(x_ref, w_ref, cnt_ref, o_ref):
    dinv = _dinv_block(cnt_ref[...])
    h = jnp.dot(x_ref[...], w_ref[...], preferred_element_type=jnp.float32)
    g = h * dinv
    o_ref[0] = g[:, :128]
    o_ref[1] = g[:, 128:]


def _tc_pre(x, W1, cnt):
    return pl.pallas_call(
        _tc_pre_body,
        grid=(_NR,),
        in_specs=[
            pl.BlockSpec((_R, 256), lambda i: (i, 0)),
            pl.BlockSpec((256, 256), lambda i: (0, 0)),
            pl.BlockSpec((2, _R, 128), lambda i: (0, i, 0)),
        ],
        out_specs=pl.BlockSpec((2, _R, 128), lambda i: (0, i, 0)),
        out_shape=jax.ShapeDtypeStruct((2, N_NODES, 128), jnp.float32),
    )(x, W1, cnt)


def _tc_mid_body(agg_ref, g_ref, cnt_ref, b_ref, w_ref, o_ref):
    dinv = _dinv_block(cnt_ref[...])
    w = w_ref[...]
    h0 = jax.nn.relu((agg_ref[0] + g_ref[0]) * dinv + b_ref[0:1, :128])
    h1 = jax.nn.relu((agg_ref[1] + g_ref[1]) * dinv + b_ref[0:1, 128:])
    h2 = (jnp.dot(h0, w[:128, :], preferred_element_type=jnp.float32)
          + jnp.dot(h1, w[128:, :], preferred_element_type=jnp.float32))
    o_ref[...] = h2 * dinv


def _tc_mid(agg1, g1, cnt, b1, W2):
    return pl.pallas_call(
        _tc_mid_body,
        grid=(_NR,),
        in_specs=[
            pl.BlockSpec((2, _R, 128), lambda i: (0, i, 0)),
            pl.BlockSpec((2, _R, 128), lambda i: (0, i, 0)),
            pl.BlockSpec((2, _R, 128), lambda i: (0, i, 0)),
            pl.BlockSpec((1, 256), lambda i: (0, 0)),
            pl.BlockSpec((256, 128), lambda i: (0, 0)),
        ],
        out_specs=pl.BlockSpec((_R, 128), lambda i: (i, 0)),
        out_shape=jax.ShapeDtypeStruct((N_NODES, 128), jnp.float32),
    )(agg1, g1, cnt, b1, W2)


def _tc_post_body(agg_ref, g_ref, cnt_ref, b_ref, o_ref):
    dinv = _dinv_block(cnt_ref[...])
    o_ref[...] = (agg_ref[0] + agg_ref[1] + g_ref[...]) * dinv + b_ref[0:1, :]


def _tc_post(agg2, g2, cnt, b2):
    return pl.pallas_call(
        _tc_post_body,
        grid=(_NR,),
        in_specs=[
            pl.BlockSpec((2, _R, 128), lambda i: (0, i, 0)),
            pl.BlockSpec((_R, 128), lambda i: (i, 0)),
            pl.BlockSpec((2, _R, 128), lambda i: (0, i, 0)),
            pl.BlockSpec((1, 128), lambda i: (0, 0)),
        ],
        out_specs=pl.BlockSpec((_R, 128), lambda i: (i, 0)),
        out_shape=jax.ShapeDtypeStruct((N_NODES, 128), jnp.float32),
    )(agg2, g2, cnt, b2)


def kernel(x, edge_index, W1, b1, W2, b2):
    src = edge_index[0].astype(jnp.int32)
    dst = edge_index[1].astype(jnp.int32)
    pad = E_PAD - N_EDGES
    src_p = jnp.concatenate([src, jnp.zeros((pad,), jnp.int32)])
    dst_p = jnp.concatenate([dst, jnp.full((pad,), N_NODES, jnp.int32)])

    src2 = jnp.stack([src_p, src_p + N_NODES]).reshape(2, 16, NCH_AGG, K)
    src2 = jnp.concatenate(
        [src2, jnp.zeros((2, 16, _PADC, K), jnp.int32)], axis=2)
    dst_agg = dst_p.reshape(16, NCH_AGG, K)
    dst_cnt = dst_p.reshape(32, NCH_CNT, K)

    src_cnt = src_p.reshape(32, NCH_CNT, K)
    src_cnt = jnp.concatenate(
        [src_cnt, jnp.zeros((32, _PADC, K), jnp.int32)], axis=1)

    ones128 = jnp.ones((K, 128), jnp.float32)
    zeros128 = jnp.zeros((R_PAD, 128), jnp.float32)

    cnt = _sc_count(dst_cnt, ones128, zeros128)[:, :N_NODES, :]

    g1 = _tc_pre(x, W1, cnt)
    agg1 = _sc_agg128(src2, dst_agg, g1.reshape(2 * N_NODES, 128), zeros128)
    g2 = _tc_mid(agg1[:, :N_NODES], g1, cnt, b1.reshape(1, 256), W2)
    agg2 = _sc_agg_esplit(src_cnt, dst_cnt, g2, zeros128)
    z = _tc_post(agg2[:, :N_NODES], g2, cnt, b2.reshape(1, 128))
    return z


# speedup vs baseline: 1.9642x; 1.0058x over previous
import functools
import jax
import jax.numpy as jnp
from jax import lax
from jax.experimental import pallas as pl
from jax.experimental.pallas import tpu as pltpu
from jax.experimental.pallas import tpu_sc as plsc

N_NODES = 10000
N_EDGES = 160000
R_PAD = 10240
E_PAD = 163840
K = 128
NCH_AGG = 80
NCH_CNT = 40
ROWS_PER_TILE = R_PAD // 16
_PADC = 8
_UNROLL = 40

_mesh = plsc.VectorSubcoreMesh(core_axis_name="c", subcore_axis_name="s")


@functools.partial(
    pl.kernel,
    out_type=jax.ShapeDtypeStruct((2, R_PAD, 128), jnp.float32),
    mesh=_mesh,
    scratch_types=[
        pltpu.VMEM((NCH_CNT, K), jnp.int32),
        pltpu.VMEM((K, 128), jnp.float32),
        pltpu.VMEM_SHARED((R_PAD, 128), jnp.float32),
    ],
)
def _sc_count(dst_hbm, ones_hbm, zeros_hbm, out_hbm, dst_v, ones_v, acc):
    cid = lax.axis_index("c")
    sid = lax.axis_index("s")
    wid = sid * 2 + cid
    rows = pl.ds(sid * ROWS_PER_TILE, ROWS_PER_TILE)

    pltpu.sync_copy(zeros_hbm.at[pl.ds(0, K)], ones_v)
    for t in range(ROWS_PER_TILE // K):
        pltpu.sync_copy(ones_v, acc.at[pl.ds(sid * ROWS_PER_TILE + t * K, K)])
    pltpu.sync_copy(ones_hbm, ones_v)
    pltpu.sync_copy(dst_hbm.at[wid], dst_v)
    plsc.subcore_barrier()

    def body(c, carry):
        pltpu.sync_copy(ones_v, acc.at[dst_v.at[c]], add=True)
        return carry

    lax.fori_loop(0, NCH_CNT, body, 0)
    plsc.subcore_barrier()
    pltpu.sync_copy(acc.at[rows], out_hbm.at[cid, rows])


def _agg_ring(nseg, table_hbm, src_v, dst_v, bufs, sems, acc):
    nb = len(bufs)

    def body(i, carry):
        c0 = i * _UNROLL
        hs = [
            pltpu.async_copy(table_hbm.at[src_v.at[c0 + b]], bufs[b], sems[b])
            for b in range(nb)
        ]
        for j in range(_UNROLL):
            b = j % nb
            hs[b].wait()
            pltpu.sync_copy(bufs[b], acc.at[dst_v.at[c0 + j]], add=True)
            if j < _UNROLL - nb:
                hs[b] = pltpu.async_copy(
                    table_hbm.at[src_v.at[c0 + j + nb]], bufs[b], sems[b])
        return carry

    lax.fori_loop(0, nseg, body, 0)


_PH = 2
_CH = NCH_AGG // _PH


@functools.partial(
    pl.kernel,
    out_type=jax.ShapeDtypeStruct((2, R_PAD, 128), jnp.float32),
    mesh=_mesh,
    scratch_types=[
        pltpu.VMEM((_CH + _PADC, K), jnp.int32),
        pltpu.VMEM((_CH, K), jnp.int32),
        pltpu.VMEM((K, 128), jnp.float32),
        pltpu.VMEM((K, 128), jnp.float32),
        pltpu.VMEM_SHARED((R_PAD, 128), jnp.float32),
        pltpu.SemaphoreType.DMA,
        pltpu.SemaphoreType.DMA,
    ],
)
def _sc_agg128(src2_hbm, dst_hbm, table_hbm, zeros_hbm, out_hbm,
               src_v, dst_v, buf0, buf1, acc, sem0, sem1):
    cid = lax.axis_index("c")
    sid = lax.axis_index("s")
    rows = pl.ds(sid * ROWS_PER_TILE, ROWS_PER_TILE)

    pltpu.sync_copy(zeros_hbm.at[pl.ds(0, K)], buf0)
    for t in range(ROWS_PER_TILE // K):
        pltpu.sync_copy(buf0, acc.at[pl.ds(sid * ROWS_PER_TILE + t * K, K)])
    plsc.subcore_barrier()

    def phase(p, carry):
        pltpu.sync_copy(src2_hbm.at[cid, sid, pl.ds(p * _CH, _CH + _PADC)],
                        src_v)
        pltpu.sync_copy(dst_hbm.at[sid, pl.ds(p * _CH, _CH)], dst_v)
        _agg_ring(_CH // _UNROLL, table_hbm, src_v, dst_v,
                  (buf0, buf1), (sem0, sem1), acc)
        return carry

    lax.fori_loop(0, _PH, phase, 0)
    plsc.subcore_barrier()
    pltpu.sync_copy(acc.at[rows], out_hbm.at[cid, rows])


@functools.partial(
    pl.kernel,
    out_type=jax.ShapeDtypeStruct((2, R_PAD, 128), jnp.float32),
    mesh=_mesh,
    scratch_types=[
        pltpu.VMEM((NCH_CNT + _PADC, K), jnp.int32),
        pltpu.VMEM((NCH_CNT, K), jnp.int32),
        pltpu.VMEM((K, 128), jnp.float32),
        pltpu.VMEM((K, 128), jnp.float32),
        pltpu.VMEM_SHARED((R_PAD, 128), jnp.float32),
        pltpu.SemaphoreType.DMA,
        pltpu.SemaphoreType.DMA,
    ],
)
def _sc_agg_esplit(src_hbm, dst_hbm, table_hbm, zeros_hbm, out_hbm,
                   src_v, dst_v, buf0, buf1, acc, sem0, sem1):
    cid = lax.axis_index("c")
    sid = lax.axis_index("s")
    wid = sid * 2 + cid
    rows = pl.ds(sid * ROWS_PER_TILE, ROWS_PER_TILE)

    pltpu.sync_copy(zeros_hbm.at[pl.ds(0, K)], buf0)
    for t in range(ROWS_PER_TILE // K):
        pltpu.sync_copy(buf0, acc.at[pl.ds(sid * ROWS_PER_TILE + t * K, K)])
    pltpu.sync_copy(src_hbm.at[wid], src_v)
    pltpu.sync_copy(dst_hbm.at[wid], dst_v)
    plsc.subcore_barrier()
    _agg_ring(NCH_CNT // _UNROLL, table_hbm, src_v, dst_v,
              (buf0, buf1), (sem0, sem1), acc)
    plsc.subcore_barrier()
    pltpu.sync_copy(acc.at[rows], out_hbm.at[cid, rows])


_R = 2000
_NR = N_NODES // _R


def _dinv_block(cnt_blk):
    deg = cnt_blk[0] + cnt_blk[1] + 1.0
    return lax.rsqrt(deg)[:, 0:1]


def _tc_pre_body(x_ref, w_ref, cnt_ref, o_ref):
    dinv = _dinv_block(cnt_ref[...])
    h = jnp.dot(x_ref[...], w_ref[...], preferred_element_type=jnp.float32)
    g = h * dinv
    o_ref[0] = g[:, :128]
    o_ref[1] = g[:, 128:]


def _tc_pre(x, W1, cnt):
    return pl.pallas_call(
        _tc_pre_body,
        grid=(_NR,),
        in_specs=[
            pl.BlockSpec((_R, 256), lambda i: (i, 0)),
            pl.BlockSpec((256, 256), lambda i: (0, 0)),
            pl.BlockSpec((2, _R, 128), lambda i: (0, i, 0)),
        ],
        out_specs=pl.BlockSpec((2, _R, 128), lambda i: (0, i, 0)),
        out_shape=jax.ShapeDtypeStruct((2, N_NODES, 128), jnp.float32),
    )(x, W1, cnt)


def _tc_mid_body(agg_ref, g_ref, cnt_ref, b_ref, w_ref, o_ref):
    dinv = _dinv_block(cnt_ref[...])
    w = w_ref[...]
    h0 = jax.nn.relu((agg_ref[0] + g_ref[0]) * dinv + b_ref[0:1, :128])
    h1 = jax.nn.relu((agg_ref[1] + g_ref[1]) * dinv + b_ref[0:1, 128:])
    h2 = (jnp.dot(h0, w[:128, :], preferred_element_type=jnp.float32)
          + jnp.dot(h1, w[128:, :], preferred_element_type=jnp.float32))
    o_ref[...] = h2 * dinv


def _tc_mid(agg1, g1, cnt, b1, W2):
    return pl.pallas_call(
        _tc_mid_body,
        grid=(_NR,),
        in_specs=[
            pl.BlockSpec((2, _R, 128), lambda i: (0, i, 0)),
            pl.BlockSpec((2, _R, 128), lambda i: (0, i, 0)),
            pl.BlockSpec((2, _R, 128), lambda i: (0, i, 0)),
            pl.BlockSpec((1, 256), lambda i: (0, 0)),
            pl.BlockSpec((256, 128), lambda i: (0, 0)),
        ],
        out_specs=pl.BlockSpec((_R, 128), lambda i: (i, 0)),
        out_shape=jax.ShapeDtypeStruct((N_NODES, 128), jnp.float32),
    )(agg1, g1, cnt, b1, W2)


def _tc_post_body(agg_ref, g_ref, cnt_ref, b_ref, o_ref):
    dinv = _dinv_block(cnt_ref[...])
    o_ref[...] = (agg_ref[0] + agg_ref[1] + g_ref[...]) * dinv + b_ref[0:1, :]


def _tc_post(agg2, g2, cnt, b2):
    return pl.pallas_call(
        _tc_post_body,
        grid=(_NR,),
        in_specs=[
            pl.BlockSpec((2, _R, 128), lambda i: (0, i, 0)),
            pl.BlockSpec((_R, 128), lambda i: (i, 0)),
            pl.BlockSpec((2, _R, 128), lambda i: (0, i, 0)),
            pl.BlockSpec((1, 128), lambda i: (0, 0)),
        ],
        out_specs=pl.BlockSpec((_R, 128), lambda i: (i, 0)),
        out_shape=jax.ShapeDtypeStruct((N_NODES, 128), jnp.float32),
    )(agg2, g2, cnt, b2)


def kernel(x, edge_index, W1, b1, W2, b2):
    src = edge_index[0].astype(jnp.int32)
    dst = edge_index[1].astype(jnp.int32)
    pad = E_PAD - N_EDGES
    src_p = jnp.concatenate([src, jnp.zeros((pad,), jnp.int32)])
    dst_p = jnp.concatenate([dst, jnp.full((pad,), N_NODES, jnp.int32)])

    src2 = jnp.stack([src_p, src_p + N_NODES]).reshape(2, 16, NCH_AGG, K)
    src2 = jnp.concatenate(
        [src2, jnp.zeros((2, 16, _PADC, K), jnp.int32)], axis=2)
    dst_agg = dst_p.reshape(16, NCH_AGG, K)
    dst_cnt = dst_p.reshape(32, NCH_CNT, K)

    src_cnt = src_p.reshape(32, NCH_CNT, K)
    src_cnt = jnp.concatenate(
        [src_cnt, jnp.zeros((32, _PADC, K), jnp.int32)], axis=1)

    ones128 = jnp.ones((K, 128), jnp.float32)
    zeros128 = jnp.zeros((R_PAD, 128), jnp.float32)

    cnt = _sc_count(dst_cnt, ones128, zeros128)[:, :N_NODES, :]

    g1 = _tc_pre(x, W1, cnt)
    agg1 = _sc_agg128(src2, dst_agg, g1.reshape(2 * N_NODES, 128), zeros128)
    g2 = _tc_mid(agg1[:, :N_NODES], g1, cnt, b1.reshape(1, 256), W2)
    agg2 = _sc_agg_esplit(src_cnt, dst_cnt, g2, zeros128)
    z = _tc_post(agg2[:, :N_NODES], g2, cnt, b2.reshape(1, 128))
    return z
